# baseline (speedup 1.0000x reference)
import jax
import jax.numpy as jnp
from jax import lax
from jax.experimental import pallas as pl
from jax.experimental.pallas import tpu as pltpu
from jax.experimental.pallas import tpu_sc as plsc

_B = 4096
_L = 200
_D = 64
_N = _B * _L
_NW = 32
_PER_W = _N // _NW
_CHUNK = 512
_CPW = _PER_W // _CHUNK
_JBLK = _CHUNK // 128
_CPL = _B // _CHUNK


def _body(idx_hbm, table_hbm, o5_hbm, idx_v, rows_v, ot_v, gsem):
    wid = lax.axis_index("s") * 2 + lax.axis_index("c")
    base_chunk = wid * _CPW

    pltpu.sync_copy(idx_hbm.at[pl.ds(wid * _PER_W, _PER_W)], idx_v)

    lane16 = lax.iota(jnp.int32, 16)

    def do_chunk(ci, carry):
        c = base_chunk + ci
        l = c // _CPL
        bblk0 = (c % _CPL) * _JBLK
        pltpu.async_copy(
            table_hbm.at[idx_v.at[pl.ds(ci * _CHUNK, _CHUNK)]], rows_v, gsem
        ).wait()
        for j in range(_JBLK):
            def do_tier(t, carry2):
                for s in range(8):
                    col = jnp.full((16,), 8 * t + s, jnp.int32)
                    for g in range(8):
                        ridx = j * 128 + g * 16 + lane16
                        vals = plsc.load_gather(rows_v, [ridx, col])
                        ot_v[s, pl.ds(g * 16, 16)] = vals
                pltpu.sync_copy(ot_v, o5_hbm.at[l, t, bblk0 + j])
                return carry2

            lax.fori_loop(0, 8, do_tier, 0)
        return carry

    lax.fori_loop(0, _CPW, do_chunk, 0)


def kernel(x, table):
    idx = x.T.reshape(_N)
    mesh = plsc.VectorSubcoreMesh(core_axis_name="c", subcore_axis_name="s")
    k = pl.kernel(
        _body,
        out_type=jax.ShapeDtypeStruct((_L, 8, _B // 128, 8, 128), jnp.float32),
        mesh=mesh,
        scratch_types=[
            pltpu.VMEM((_PER_W,), jnp.int32),
            pltpu.VMEM((_CHUNK, _D), jnp.float32),
            pltpu.VMEM((8, 128), jnp.float32),
            pltpu.SemaphoreType.DMA,
        ],
        compiler_params=pltpu.CompilerParams(
            use_tc_tiling_on_sc=False, needs_layout_passes=False
        ),
    )
    o5 = k(idx, table)
    return o5.transpose((2, 4, 0, 1, 3)).reshape(_B, _L, _D)


# speedup vs baseline: 2.0942x; 2.0942x over previous
import jax
import jax.numpy as jnp
from jax import lax
from jax.experimental import pallas as pl
from jax.experimental.pallas import tpu as pltpu
from jax.experimental.pallas import tpu_sc as plsc

_B = 4096
_L = 200
_D = 64
_N = _B * _L
_NW = 32
_PER_W = _N // _NW
_CHUNK = 512
_CPW = _PER_W // _CHUNK
_JBLK = _CHUNK // 128
_CPL = _B // _CHUNK
_OTP = 129


def _body(
    idx_hbm, table_hbm, o5_hbm,
    idx_v, rows0, rows1, ot0, ot1,
    gsem0, gsem1, wsem0, wsem1,
):
    wid = lax.axis_index("s") * 2 + lax.axis_index("c")
    base_chunk = wid * _CPW

    pltpu.sync_copy(idx_hbm.at[pl.ds(wid * _PER_W, _PER_W)], idx_v)

    lane16 = lax.iota(jnp.int32, 16)
    tsg = []
    for g in range(_D // 16):
        d = g * 16 + lane16
        tsg.append((d >> 3, d & 7))

    rows = (rows0, rows1)
    gsems = (gsem0, gsem1)
    ots = (ot0, ot1)
    wsems = (wsem0, wsem1)

    def gather_copy(ci, p):
        return pltpu.make_async_copy(
            table_hbm.at[idx_v.at[pl.ds(ci * _CHUNK, _CHUNK)]],
            rows[p],
            gsems[p],
        )

    def write_copy(l, bj, q):
        return pltpu.make_async_copy(
            ots[q].at[:, :, pl.ds(0, 128)],
            o5_hbm.at[l, :, bj],
            wsems[q],
        )

    def process_chunk(ci, p):
        c = base_chunk + ci
        l = c // _CPL
        bblk0 = (c % _CPL) * _JBLK
        for j in range(_JBLK):
            q = j % 2
            m = ci * _JBLK + j

            @pl.when(m >= 2)
            def _wait_prev():
                write_copy(l, bblk0 + j, q).wait()

            def rowblk(rb, carry2):
                for u in range(8):
                    rr = rb * 8 + u
                    lane_b = jnp.full((16,), rr, jnp.int32)
                    r = j * 128 + rr
                    for g in range(_D // 16):
                        vals = rows[p][r, pl.ds(g * 16, 16)]
                        plsc.store_scatter(
                            ots[q], [tsg[g][0], tsg[g][1], lane_b], vals
                        )
                return carry2

            lax.fori_loop(0, 16, rowblk, 0)
            write_copy(l, bblk0 + j, q).start()
        return l

    gather_copy(0, 0).start()

    def two_chunks(h, carry):
        for p in range(2):
            ci = 2 * h + p
            nci = ci + 1

            @pl.when(nci < _CPW)
            def _start_next():
                gather_copy(nci, (p + 1) % 2).start()

            gather_copy(ci, p).wait()
            process_chunk(ci, p)
        return carry

    lax.fori_loop(0, _CPW // 2, two_chunks, 0)

    write_copy(_L - 1, _B // 128 - 1, 0).wait()
    write_copy(_L - 1, _B // 128 - 1, 1).wait()


def kernel(x, table):
    idx = x.T.reshape(_N)
    mesh = plsc.VectorSubcoreMesh(core_axis_name="c", subcore_axis_name="s")
    k = pl.kernel(
        _body,
        out_type=jax.ShapeDtypeStruct((_L, 8, _B // 128, 8, 128), jnp.float32),
        mesh=mesh,
        scratch_types=[
            pltpu.VMEM((_PER_W,), jnp.int32),
            pltpu.VMEM((_CHUNK, _D), jnp.float32),
            pltpu.VMEM((_CHUNK, _D), jnp.float32),
            pltpu.VMEM((8, 8, _OTP), jnp.float32),
            pltpu.VMEM((8, 8, _OTP), jnp.float32),
            pltpu.SemaphoreType.DMA,
            pltpu.SemaphoreType.DMA,
            pltpu.SemaphoreType.DMA,
            pltpu.SemaphoreType.DMA,
        ],
        compiler_params=pltpu.CompilerParams(
            use_tc_tiling_on_sc=False, needs_layout_passes=False
        ),
    )
    o5 = k(idx, table)
    return o5.transpose((2, 4, 0, 1, 3)).reshape(_B, _L, _D)


# speedup vs baseline: 2.6646x; 1.2724x over previous
import jax
import jax.numpy as jnp
from jax import lax
from jax.experimental import pallas as pl
from jax.experimental.pallas import tpu as pltpu
from jax.experimental.pallas import tpu_sc as plsc

_B = 4096
_L = 200
_D = 64
_N = _B * _L
_NW = 32
_PER_W = _N // _NW
_CHUNK = 512
_CPW = _PER_W // _CHUNK
_JBLK = _CHUNK // 128
_CPL = _B // _CHUNK
_OTP = 129


def _body(
    idx_hbm, table_hbm, o5_hbm,
    idx_v, rows0, rows1, ot0, ot1,
    gsem0, gsem1, wsem0, wsem1,
):
    wid = lax.axis_index("s") * 2 + lax.axis_index("c")
    base_chunk = wid * _CPW

    pltpu.sync_copy(idx_hbm.at[pl.ds(wid * _PER_W, _PER_W)], idx_v)

    lane16 = lax.iota(jnp.int32, 16)
    tsg = []
    for g in range(_D // 16):
        d = g * 16 + lane16
        tsg.append((d >> 3, d & 7))

    rows = (rows0, rows1)
    gsems = (gsem0, gsem1)
    ots = (ot0, ot1)
    wsems = (wsem0, wsem1)

    def gather_copy(ci, p):
        return pltpu.make_async_copy(
            table_hbm.at[idx_v.at[pl.ds(ci * _CHUNK, _CHUNK)]],
            rows[p],
            gsems[p],
        )

    def write_copy(l, bj, q):
        return pltpu.make_async_copy(
            ots[q].at[:, :, pl.ds(0, 128)],
            o5_hbm.at[l, :, bj],
            wsems[q],
        )

    def process_chunk(ci, p):
        c = base_chunk + ci
        l = c // _CPL
        bblk0 = (c % _CPL) * _JBLK
        for j in range(_JBLK):
            q = j % 2
            m = ci * _JBLK + j

            @pl.when(m >= 2)
            def _wait_prev():
                write_copy(l, bblk0 + j, q).wait()

            @plsc.parallel_loop(0, 128, step=8)
            def _rowblk(rr0):
                for u in range(8):
                    rr = rr0 + u
                    lane_b = jnp.full((16,), rr, jnp.int32)
                    r = j * 128 + rr
                    for g in range(_D // 16):
                        vals = rows[p][r, pl.ds(g * 16, 16)]
                        plsc.store_scatter(
                            ots[q], [tsg[g][0], tsg[g][1], lane_b], vals
                        )
            write_copy(l, bblk0 + j, q).start()
        return l

    gather_copy(0, 0).start()

    def two_chunks(h, carry):
        for p in range(2):
            ci = 2 * h + p
            nci = ci + 1

            @pl.when(nci < _CPW)
            def _start_next():
                gather_copy(nci, (p + 1) % 2).start()

            gather_copy(ci, p).wait()
            process_chunk(ci, p)
        return carry

    lax.fori_loop(0, _CPW // 2, two_chunks, 0)

    write_copy(_L - 1, _B // 128 - 1, 0).wait()
    write_copy(_L - 1, _B // 128 - 1, 1).wait()


def kernel(x, table):
    idx = x.T.reshape(_N)
    mesh = plsc.VectorSubcoreMesh(core_axis_name="c", subcore_axis_name="s")
    k = pl.kernel(
        _body,
        out_type=jax.ShapeDtypeStruct((_L, 8, _B // 128, 8, 128), jnp.float32),
        mesh=mesh,
        scratch_types=[
            pltpu.VMEM((_PER_W,), jnp.int32),
            pltpu.VMEM((_CHUNK, _D), jnp.float32),
            pltpu.VMEM((_CHUNK, _D), jnp.float32),
            pltpu.VMEM((8, 8, _OTP), jnp.float32),
            pltpu.VMEM((8, 8, _OTP), jnp.float32),
            pltpu.SemaphoreType.DMA,
            pltpu.SemaphoreType.DMA,
            pltpu.SemaphoreType.DMA,
            pltpu.SemaphoreType.DMA,
        ],
        compiler_params=pltpu.CompilerParams(
            use_tc_tiling_on_sc=False,
            needs_layout_passes=False,
            disable_bounds_checks=True,
        ),
    )
    o5 = k(idx, table)
    return o5.transpose((2, 4, 0, 1, 3)).reshape(_B, _L, _D)
